# baseline (speedup 1.0000x reference)
import functools

import jax
import jax.numpy as jnp
from jax import lax
from jax.experimental import pallas as pl
from jax.experimental.pallas import tpu as pltpu
from jax.experimental.pallas import tpu_sc as plsc

N = 10000
E = 320000
D = 128
G = 64
OUT = 10

NC = 2
NS = 16
CH = 80

E_PER_SC = E // NC
NCH = E // (NC * NS * CH)
RPT = 624
RPT_LAST = N - RPT * (NS - 1)
ZR = 208


def _segment_sum_sc(x, src, dst3, zrows):
    mesh = plsc.VectorSubcoreMesh(core_axis_name="c", subcore_axis_name="s")

    @functools.partial(
        pl.kernel,
        mesh=mesh,
        out_type=jax.ShapeDtypeStruct((NC, N, D), jnp.float32),
        scratch_types=[
            pltpu.VMEM((NCH * CH,), jnp.int32),
            pltpu.VMEM((NCH, 1, CH), jnp.int32),
            [pltpu.VMEM((CH, D), jnp.float32) for _ in range(2)],
            pltpu.VMEM_SHARED((N, D), jnp.float32),
            [pltpu.SemaphoreType.DMA for _ in range(2)],
            [pltpu.SemaphoreType.DMA for _ in range(2)],
            pltpu.SemaphoreType.DMA,
        ],
    )
    def k(x_hbm, src_hbm, dst3_hbm, z_hbm, out_hbm, src_all, dst_all, buf,
          acc, sg, ss, sz):
        cid = lax.axis_index("c")
        sid = lax.axis_index("s")

        base_e = cid * E_PER_SC + sid * (NCH * CH)
        base_c = base_e // CH

        def start_gather(i, slot):
            pltpu.async_copy(x_hbm.at[src_all.at[pl.ds(i * CH, CH)]],
                             buf[slot], sg[slot])

        def wait_gather(slot):
            pltpu.make_async_copy(x_hbm.at[pl.ds(0, CH)], buf[slot],
                                  sg[slot]).wait()

        def start_scatter(i, slot):
            pltpu.async_copy(buf[slot], acc.at[dst_all.at[i, 0]], ss[slot],
                             add=True)

        def wait_scatter(slot):
            pltpu.make_async_copy(buf[slot], acc.at[dst_all.at[0, 0]],
                                  ss[slot]).wait()

        pltpu.sync_copy(src_hbm.at[pl.ds(base_e, NCH * CH)], src_all)
        pltpu.sync_copy(dst3_hbm.at[pl.ds(base_c, NCH)], dst_all)
        start_gather(0, 0)

        rbase = sid * RPT
        for j in range(RPT // ZR):
            pltpu.async_copy(z_hbm, acc.at[pl.ds(rbase + j * ZR, ZR)], sz)

        @pl.when(sid == NS - 1)
        def _():
            pltpu.async_copy(z_hbm.at[pl.ds(0, RPT_LAST - RPT)],
                             acc.at[pl.ds(rbase + RPT, RPT_LAST - RPT)], sz)

        for j in range(RPT // ZR):
            pltpu.make_async_copy(z_hbm, acc.at[pl.ds(rbase, ZR)], sz).wait()

        @pl.when(sid == NS - 1)
        def _():
            pltpu.make_async_copy(z_hbm.at[pl.ds(0, RPT_LAST - RPT)],
                                  acc.at[pl.ds(rbase, RPT_LAST - RPT)],
                                  sz).wait()

        plsc.subcore_barrier()

        def pair(g, carry):
            for half in (0, 1):
                i = g * 2 + half
                wait_gather(half)

                @pl.when(i >= 1)
                def _():
                    wait_scatter(1 - half)

                start_gather(i + 1, 1 - half)
                start_scatter(i, half)
            return carry

        lax.fori_loop(0, (NCH - 1) // 2, pair, 0)
        wait_gather(0)
        wait_scatter(1)
        start_scatter(NCH - 1, 0)
        wait_scatter(0)

        plsc.subcore_barrier()

        @pl.when(sid < NS - 1)
        def _():
            pltpu.sync_copy(
                acc.at[pl.ds(rbase, RPT)],
                out_hbm.at[cid, pl.ds(rbase, RPT)],
            )

        @pl.when(sid == NS - 1)
        def _():
            pltpu.sync_copy(
                acc.at[pl.ds(rbase, RPT_LAST)],
                out_hbm.at[cid, pl.ds(rbase, RPT_LAST)],
            )

    return k(x, src, dst3, zrows)


def _gconv_dense_tc(partials, x, Wrel, Wroot, brel):
    RB = 1000

    def body(p_ref, x_ref, wr_ref, wo_ref, b_ref, o_ref):
        agg = p_ref[0] + p_ref[1]
        h = jnp.dot(agg, wr_ref[...], preferred_element_type=jnp.float32)
        h = h + jnp.dot(x_ref[...], wo_ref[...], preferred_element_type=jnp.float32)
        o_ref[...] = jnp.maximum(h + b_ref[...], 0.0)

    return pl.pallas_call(
        body,
        grid=(N // RB,),
        in_specs=[
            pl.BlockSpec((NC, RB, D), lambda i: (0, i, 0)),
            pl.BlockSpec((RB, D), lambda i: (i, 0)),
            pl.BlockSpec((D, D), lambda i: (0, 0)),
            pl.BlockSpec((D, D), lambda i: (0, 0)),
            pl.BlockSpec((1, D), lambda i: (0, 0)),
        ],
        out_specs=pl.BlockSpec((RB, D), lambda i: (i, 0)),
        out_shape=jax.ShapeDtypeStruct((N, D), jnp.float32),
    )(partials, x, Wrel, Wroot, brel.reshape(1, D))


def _gconv3_classifier_tc(partials, x, Wrel, Wroot, brel,
                          Wlin1, blin1, batch3, Wfin):
    RB = 1000
    GRID = N // RB

    def body(p_ref, x_ref, wr_ref, wo_ref, b_ref, b3_ref, w1_ref, b1_ref,
             wf_ref, o_ref, acc_ref):
        i = pl.program_id(0)

        @pl.when(i == 0)
        def _():
            acc_ref[...] = jnp.zeros_like(acc_ref)

        agg = p_ref[0] + p_ref[1]
        h = jnp.dot(agg, wr_ref[...], preferred_element_type=jnp.float32)
        h = h + jnp.dot(x_ref[...], wo_ref[...],
                        preferred_element_type=jnp.float32)
        h = jnp.maximum(h + b_ref[...], 0.0)
        z = jnp.dot(h, w1_ref[...], preferred_element_type=jnp.float32)
        z = jnp.maximum(z + b1_ref[...], 0.0)
        segs = lax.broadcasted_iota(jnp.int32, (G, RB), 0)
        oh = (segs == b3_ref[0]).astype(jnp.float32)
        acc_ref[:, :D] += jnp.dot(oh, z, preferred_element_type=jnp.float32)
        cnt = jnp.sum(oh, axis=1)
        col = lax.broadcasted_iota(jnp.int32, (G, 16), 1)
        acc_ref[:, D:] += jnp.where(col == 0, cnt[:, None], 0.0)

        @pl.when(i == GRID - 1)
        def _():
            o_ref[...] = jnp.dot(
                acc_ref[...], wf_ref[...], preferred_element_type=jnp.float32
            )

    return pl.pallas_call(
        body,
        grid=(GRID,),
        in_specs=[
            pl.BlockSpec((NC, RB, D), lambda i: (0, i, 0)),
            pl.BlockSpec((RB, D), lambda i: (i, 0)),
            pl.BlockSpec((D, D), lambda i: (0, 0)),
            pl.BlockSpec((D, D), lambda i: (0, 0)),
            pl.BlockSpec((1, D), lambda i: (0, 0)),
            pl.BlockSpec((1, 1, RB), lambda i: (i, 0, 0)),
            pl.BlockSpec((D, D), lambda i: (0, 0)),
            pl.BlockSpec((1, D), lambda i: (0, 0)),
            pl.BlockSpec((D + 16, D), lambda i: (0, 0)),
        ],
        out_specs=pl.BlockSpec((G, D), lambda i: (0, 0)),
        out_shape=jax.ShapeDtypeStruct((G, D), jnp.float32),
        scratch_shapes=[pltpu.VMEM((G, D + 16), jnp.float32)],
    )(partials, x, Wrel, Wroot, brel.reshape(1, D), batch3,
      Wlin1, blin1.reshape(1, D), Wfin)


def kernel(x, edge_index, batch,
           Wrel0, brel0, Wroot0,
           Wrel1, brel1, Wroot1,
           Wrel2, brel2, Wroot2,
           Wlin1, blin1, Wlin2, blin2):
    src = edge_index[0]
    dst3 = edge_index[1].reshape(E // CH, 1, CH)
    zrows = jnp.zeros((ZR, D), jnp.float32)

    h = x
    for Wrel, brel, Wroot in (
        (Wrel0, brel0, Wroot0),
        (Wrel1, brel1, Wroot1),
    ):
        partials = _segment_sum_sc(h, src, dst3, zrows)
        h = _gconv_dense_tc(partials, h, Wrel, Wroot, brel)

    Wfin = jnp.zeros((D + 16, D), jnp.float32)
    Wfin = Wfin.at[:D, :OUT].set(Wlin2)
    Wfin = Wfin.at[D, :OUT].set(blin2)
    batch3 = batch.reshape(N // 1000, 1, 1000)

    partials = _segment_sum_sc(h, src, dst3, zrows)
    out = _gconv3_classifier_tc(partials, h, Wrel2, Wroot2, brel2,
                                Wlin1, blin1, batch3, Wfin)
    return out[:, :OUT]


# speedup vs baseline: 1.4405x; 1.4405x over previous
import functools

import jax
import jax.numpy as jnp
from jax import lax
from jax.experimental import pallas as pl
from jax.experimental.pallas import tpu as pltpu
from jax.experimental.pallas import tpu_sc as plsc

N = 10000
E = 320000
D = 128
G = 64
OUT = 10

NC = 2
NS = 16
CH = 80

E_PER_SC = E // NC
NCH = E // (NC * NS * CH)
RPT = 624
RPT_LAST = N - RPT * (NS - 1)
ZR = 208


def _segment_sum_sc(x, src, dst3, zrows):
    mesh = plsc.VectorSubcoreMesh(core_axis_name="c", subcore_axis_name="s")

    @functools.partial(
        pl.kernel,
        mesh=mesh,
        out_type=jax.ShapeDtypeStruct((NC, N, D), jnp.float32),
        scratch_types=[
            pltpu.VMEM((NCH * CH,), jnp.int32),
            [pltpu.VMEM((CH,), jnp.int32) for _ in range(3)],
            [pltpu.VMEM((CH, D), jnp.float32) for _ in range(3)],
            pltpu.VMEM_SHARED((N, D), jnp.float32),
            [pltpu.SemaphoreType.DMA for _ in range(3)],
            [pltpu.SemaphoreType.DMA for _ in range(3)],
            pltpu.SemaphoreType.DMA,
        ],
    )
    def k(x_hbm, src_hbm, dst3_hbm, z_hbm, out_hbm, src_all, dv, buf,
          acc, sg, ss, sz):
        cid = lax.axis_index("c")
        sid = lax.axis_index("s")

        base_e = cid * E_PER_SC + sid * (NCH * CH)
        base_c = base_e // CH

        def start_chunk(i, slot):
            pltpu.async_copy(dst3_hbm.at[base_c + i, 0], dv[slot], sg[slot])
            pltpu.async_copy(x_hbm.at[src_all.at[pl.ds(i * CH, CH)]],
                             buf[slot], sg[slot])

        def wait_chunk(slot):
            pltpu.make_async_copy(dst3_hbm.at[0, 0], dv[slot], sg[slot]).wait()
            pltpu.make_async_copy(x_hbm.at[pl.ds(0, CH)], buf[slot],
                                  sg[slot]).wait()

        def start_scatter(i, slot):
            pltpu.async_copy(buf[slot], acc.at[dv[slot]], ss[slot],
                             add=True)

        def wait_scatter(slot):
            pltpu.make_async_copy(buf[slot], acc.at[dv[slot]],
                                  ss[slot]).wait()

        pltpu.sync_copy(src_hbm.at[pl.ds(base_e, NCH * CH)], src_all)
        start_chunk(0, 0)
        start_chunk(1, 1)

        rbase = sid * RPT
        for j in range(RPT // ZR):
            pltpu.async_copy(z_hbm, acc.at[pl.ds(rbase + j * ZR, ZR)], sz)

        @pl.when(sid == NS - 1)
        def _():
            pltpu.async_copy(z_hbm.at[pl.ds(0, RPT_LAST - RPT)],
                             acc.at[pl.ds(rbase + RPT, RPT_LAST - RPT)], sz)

        for j in range(RPT // ZR):
            pltpu.make_async_copy(z_hbm, acc.at[pl.ds(rbase, ZR)], sz).wait()

        @pl.when(sid == NS - 1)
        def _():
            pltpu.make_async_copy(z_hbm.at[pl.ds(0, RPT_LAST - RPT)],
                                  acc.at[pl.ds(rbase, RPT_LAST - RPT)],
                                  sz).wait()

        plsc.subcore_barrier()

        def tri(g, carry):
            for h in (0, 1, 2):
                i = g * 3 + h

                @pl.when(i < NCH)
                def _():
                    wait_chunk(h)

                @pl.when(jnp.logical_and(i >= 1, i <= NCH))
                def _():
                    wait_scatter((h + 2) % 3)

                @pl.when(i + 2 < NCH)
                def _():
                    start_chunk(i + 2, (h + 2) % 3)

                @pl.when(i < NCH)
                def _():
                    start_scatter(i, h)
            return carry

        lax.fori_loop(0, (NCH + 3) // 3, tri, 0)

        plsc.subcore_barrier()

        @pl.when(sid < NS - 1)
        def _():
            pltpu.sync_copy(
                acc.at[pl.ds(rbase, RPT)],
                out_hbm.at[cid, pl.ds(rbase, RPT)],
            )

        @pl.when(sid == NS - 1)
        def _():
            pltpu.sync_copy(
                acc.at[pl.ds(rbase, RPT_LAST)],
                out_hbm.at[cid, pl.ds(rbase, RPT_LAST)],
            )

    return k(x, src, dst3, zrows)


def _gconv_dense_tc(partials, x, Wrel, Wroot, brel):
    RB = 1000

    def body(p_ref, x_ref, wr_ref, wo_ref, b_ref, o_ref):
        agg = p_ref[0] + p_ref[1]
        h = jnp.dot(agg, wr_ref[...], preferred_element_type=jnp.float32)
        h = h + jnp.dot(x_ref[...], wo_ref[...], preferred_element_type=jnp.float32)
        o_ref[...] = jnp.maximum(h + b_ref[...], 0.0)

    return pl.pallas_call(
        body,
        grid=(N // RB,),
        in_specs=[
            pl.BlockSpec((NC, RB, D), lambda i: (0, i, 0)),
            pl.BlockSpec((RB, D), lambda i: (i, 0)),
            pl.BlockSpec((D, D), lambda i: (0, 0)),
            pl.BlockSpec((D, D), lambda i: (0, 0)),
            pl.BlockSpec((1, D), lambda i: (0, 0)),
        ],
        out_specs=pl.BlockSpec((RB, D), lambda i: (i, 0)),
        out_shape=jax.ShapeDtypeStruct((N, D), jnp.float32),
    )(partials, x, Wrel, Wroot, brel.reshape(1, D))


def _gconv3_classifier_tc(partials, x, Wrel, Wroot, brel,
                          Wlin1, blin1, batch3, Wfin):
    RB = 1000
    GRID = N // RB

    def body(p_ref, x_ref, wr_ref, wo_ref, b_ref, b3_ref, w1_ref, b1_ref,
             wf_ref, o_ref, acc_ref):
        i = pl.program_id(0)

        @pl.when(i == 0)
        def _():
            acc_ref[...] = jnp.zeros_like(acc_ref)

        agg = p_ref[0] + p_ref[1]
        h = jnp.dot(agg, wr_ref[...], preferred_element_type=jnp.float32)
        h = h + jnp.dot(x_ref[...], wo_ref[...],
                        preferred_element_type=jnp.float32)
        h = jnp.maximum(h + b_ref[...], 0.0)
        z = jnp.dot(h, w1_ref[...], preferred_element_type=jnp.float32)
        z = jnp.maximum(z + b1_ref[...], 0.0)
        segs = lax.broadcasted_iota(jnp.int32, (G, RB), 0)
        oh = (segs == b3_ref[0]).astype(jnp.float32)
        acc_ref[:, :D] += jnp.dot(oh, z, preferred_element_type=jnp.float32)
        cnt = jnp.sum(oh, axis=1)
        col = lax.broadcasted_iota(jnp.int32, (G, 16), 1)
        acc_ref[:, D:] += jnp.where(col == 0, cnt[:, None], 0.0)

        @pl.when(i == GRID - 1)
        def _():
            o_ref[...] = jnp.dot(
                acc_ref[...], wf_ref[...], preferred_element_type=jnp.float32
            )

    return pl.pallas_call(
        body,
        grid=(GRID,),
        in_specs=[
            pl.BlockSpec((NC, RB, D), lambda i: (0, i, 0)),
            pl.BlockSpec((RB, D), lambda i: (i, 0)),
            pl.BlockSpec((D, D), lambda i: (0, 0)),
            pl.BlockSpec((D, D), lambda i: (0, 0)),
            pl.BlockSpec((1, D), lambda i: (0, 0)),
            pl.BlockSpec((1, 1, RB), lambda i: (i, 0, 0)),
            pl.BlockSpec((D, D), lambda i: (0, 0)),
            pl.BlockSpec((1, D), lambda i: (0, 0)),
            pl.BlockSpec((D + 16, D), lambda i: (0, 0)),
        ],
        out_specs=pl.BlockSpec((G, D), lambda i: (0, 0)),
        out_shape=jax.ShapeDtypeStruct((G, D), jnp.float32),
        scratch_shapes=[pltpu.VMEM((G, D + 16), jnp.float32)],
    )(partials, x, Wrel, Wroot, brel.reshape(1, D), batch3,
      Wlin1, blin1.reshape(1, D), Wfin)


def kernel(x, edge_index, batch,
           Wrel0, brel0, Wroot0,
           Wrel1, brel1, Wroot1,
           Wrel2, brel2, Wroot2,
           Wlin1, blin1, Wlin2, blin2):
    src = edge_index[0]
    dst3 = edge_index[1].reshape(E // CH, 1, CH)
    zrows = jnp.zeros((ZR, D), jnp.float32)

    h = x
    for Wrel, brel, Wroot in (
        (Wrel0, brel0, Wroot0),
        (Wrel1, brel1, Wroot1),
    ):
        partials = _segment_sum_sc(h, src, dst3, zrows)
        h = _gconv_dense_tc(partials, h, Wrel, Wroot, brel)

    Wfin = jnp.zeros((D + 16, D), jnp.float32)
    Wfin = Wfin.at[:D, :OUT].set(Wlin2)
    Wfin = Wfin.at[D, :OUT].set(blin2)
    batch3 = batch.reshape(N // 1000, 1, 1000)

    partials = _segment_sum_sc(h, src, dst3, zrows)
    out = _gconv3_classifier_tc(partials, h, Wrel2, Wroot2, brel2,
                                Wlin1, blin1, batch3, Wfin)
    return out[:, :OUT]
